# baseline (speedup 1.0000x reference)
import jax
import jax.numpy as jnp
from jax import lax
from jax.experimental import pallas as pl
from jax.experimental.pallas import tpu as pltpu

_BM = 1024
_NBUF = 4
_CHUNK = 256


def _dot(a, b):
    return jax.lax.dot_general(
        a, b, dimension_numbers=(((1,), (0,)), ((), ())),
        precision=jax.lax.Precision.DEFAULT,
        preferred_element_type=jnp.float32)


def _make_body(nsteps):
    def body(x_hbm, w1_ref, b1_ref, w2_ref, b2_ref, out_hbm,
             xbuf, obuf, insems, outsems):
        def in_cp(i, slot):
            return pltpu.make_async_copy(
                x_hbm.at[pl.ds(i * _BM, _BM), :],
                xbuf.at[slot],
                insems.at[slot])

        def out_cp(i):
            return pltpu.make_async_copy(
                obuf.at[i],
                out_hbm.at[pl.ds(i * _BM, _BM), :],
                outsems.at[i])

        w1 = w1_ref[...]
        b1 = b1_ref[...]
        w2 = w2_ref[...]
        b2 = b2_ref[...]

        for i in range(_NBUF - 1):
            in_cp(i, i).start()

        def step(i, carry):
            slot = lax.rem(i, _NBUF)
            in_cp(i, slot).wait()
            for j in range(_BM // _CHUNK):
                xc = xbuf[slot, pl.ds(j * _CHUNK, _CHUNK), :]
                h = jnp.maximum(_dot(xc, w1) + b1, 0.0)
                obuf[i, pl.ds(j * _CHUNK, _CHUNK), :] = _dot(h, w2) + b2
            out_cp(i).start()

            nxt = i + _NBUF - 1
            nslot = lax.rem(nxt, _NBUF)

            @pl.when(nxt < nsteps)
            def _():
                in_cp(nxt, nslot).start()
            return carry

        lax.fori_loop(0, nsteps, step, 0)
        for i in range(nsteps):
            out_cp(i).wait()
    return body


def kernel(inputs, enc_w1, enc_b1, enc_w2, enc_b2,
           dec_w1, dec_b1, dec_w2, dec_b2, prior):
    del dec_w1, dec_b1, dec_w2, dec_b2, prior
    b, feat = inputs.shape
    hid = enc_w1.shape[1]
    code = enc_w2.shape[1]
    nsteps = b // _BM
    out = pl.pallas_call(
        _make_body(nsteps),
        in_specs=[
            pl.BlockSpec(memory_space=pl.ANY),
            pl.BlockSpec(memory_space=pltpu.VMEM),
            pl.BlockSpec(memory_space=pltpu.VMEM),
            pl.BlockSpec(memory_space=pltpu.VMEM),
            pl.BlockSpec(memory_space=pltpu.VMEM),
        ],
        out_specs=pl.BlockSpec(memory_space=pl.ANY),
        out_shape=jax.ShapeDtypeStruct((b, code), jnp.float32),
        scratch_shapes=[
            pltpu.VMEM((_NBUF, _BM, feat), jnp.float32),
            pltpu.VMEM((nsteps, _BM, code), jnp.float32),
            pltpu.SemaphoreType.DMA((_NBUF,)),
            pltpu.SemaphoreType.DMA((nsteps,)),
        ],
    )(inputs, enc_w1, enc_b1.reshape(1, hid),
      enc_w2, enc_b2.reshape(1, code))
    return out
